# baseline (speedup 1.0000x reference)
import functools

import jax
import jax.numpy as jnp
from jax import lax
from jax.experimental import pallas as pl
from jax.experimental.pallas import tpu as pltpu
from jax.experimental.pallas import tpu_sc as plsc

N = 10000
E = 320000
D_IN = 128
H = 256
C = 8

NP = 10240
NC = 2
NS = 16
TILES = NC * NS
PT = NP // TILES
CAP = 16384
K = 32
CH = 4000
_BN_C = 1.0 / (1.0 + 1e-5) ** 0.5


def _mesh():
    return plsc.VectorSubcoreMesh(core_axis_name="c", subcore_axis_name="s")


def _sc_prologue(edge):
    @functools.partial(
        pl.kernel,
        out_type=(
            jax.ShapeDtypeStruct((NP,), jnp.float32),
            jax.ShapeDtypeStruct((TILES, CAP), jnp.int32),
            jax.ShapeDtypeStruct((TILES, 16), jnp.int32),
        ),
        mesh=_mesh(),
        compiler_params=pltpu.CompilerParams(needs_layout_passes=False),
        scratch_types=[
            pltpu.VMEM((CH,), jnp.int32),
            pltpu.VMEM((CH,), jnp.int32),
            pltpu.VMEM((CAP,), jnp.int32),
            pltpu.VMEM((PT,), jnp.float32),
            pltpu.VMEM((16,), jnp.int32),
        ],
    )
    def prologue(edge_hbm, deg_hbm, pkl_hbm, cnt_hbm,
                 srcb, dstb, pkl, degl, cntv):
        cid = lax.axis_index("c")
        sid = lax.axis_index("s")
        wid = sid * NC + cid
        lo = wid * PT

        zf = jnp.zeros((16,), jnp.float32)

        def zdeg(i, carry):
            degl[pl.ds(i * 16, 16)] = zf
            return carry

        lax.fori_loop(0, PT // 16, zdeg, 0)

        lanes = lax.iota(jnp.int32, 16)

        def zl(i, carry):
            pkl[pl.ds(i * 16, 16)] = PT * 16384 + ((i * 16 + lanes) & 8191)
            return carry

        lax.fori_loop(0, CAP // 16, zl, 0)

        ones = jnp.ones((16,), jnp.float32)

        def chunk_body(ci, cnt):
            base = ci * CH
            pltpu.sync_copy(edge_hbm.at[pl.ds(base, CH)], srcb)
            pltpu.sync_copy(edge_hbm.at[pl.ds(E + base, CH)], dstb)

            def vec_body(j, cnt):
                sv = srcb[pl.ds(j * 16, 16)]
                dv = dstb[pl.ds(j * 16, 16)]
                dl = dv - lo
                m = (dl >= 0) & (dl < PT)
                mi = m.astype(jnp.int32)
                pos = cnt + plsc.cumsum(mi) - 1
                plsc.store_scatter(pkl, [pos], dl * 16384 + sv, mask=m)
                plsc.addupdate_scatter(degl, [dl], ones, mask=m)
                return pos[15] + 1

            return lax.fori_loop(0, CH // 16, vec_body, cnt)

        cnt = lax.fori_loop(0, E // CH, chunk_body, jnp.int32(0))
        cntv[...] = jnp.where(lax.iota(jnp.int32, 16) == 0, cnt, 0)
        pltpu.sync_copy(degl, deg_hbm.at[pl.ds(lo, PT)])
        pltpu.sync_copy(pkl, pkl_hbm.at[wid])
        pltpu.sync_copy(cntv, cnt_hbm.at[wid])

    return prologue(edge.reshape(2 * E))


def _sc_layer(y, pkl, cnts):
    @functools.partial(
        pl.kernel,
        out_type=jax.ShapeDtypeStruct((NP, H), jnp.float32),
        mesh=_mesh(),
        compiler_params=pltpu.CompilerParams(needs_layout_passes=False),
        scratch_types=[
            pltpu.VMEM((PT + 1, H), jnp.float32),
            pltpu.VMEM((CAP,), jnp.int32),
            pltpu.VMEM((2, K), jnp.int32),
            pltpu.VMEM((2, K, H), jnp.float32),
            pltpu.VMEM((16,), jnp.int32),
            pltpu.SemaphoreType.DMA((2,)),
        ],
    )
    def layer(y_hbm, pkl_hbm, cnt_hbm, out_hbm,
              acc, pkl, sidx, rows, cntv, sems):
        cid = lax.axis_index("c")
        sid = lax.axis_index("s")
        wid = sid * NC + cid
        lo = wid * PT

        pltpu.sync_copy(cnt_hbm.at[wid], cntv)
        pltpu.sync_copy(pkl_hbm.at[wid], pkl)
        cnt = jnp.sum(cntv[...])
        nch = (cnt + (K - 1)) // K

        zf = jnp.zeros((16,), jnp.float32)

        def zacc(r, carry):
            for u in range(H // 16):
                acc[r, pl.ds(u * 16, 16)] = zf
            return carry

        lax.fori_loop(0, PT + 1, zacc, 0)

        def issue(i, b):
            for g in range(K // 16):
                pkv = pkl[pl.ds(i * K + g * 16, 16)]
                sidx[b, pl.ds(g * 16, 16)] = pkv & 16383
            pltpu.async_copy(y_hbm.at[sidx.at[b]], rows.at[b], sems.at[b])

        @pl.when(nch > 0)
        def _():
            issue(0, 0)

        def chunk(i, carry):
            b = lax.rem(i, 2)

            @pl.when(i + 1 < nch)
            def _():
                issue(i + 1, 1 - b)

            pltpu.make_async_copy(
                y_hbm.at[sidx.at[b]], rows.at[b], sems.at[b]
            ).wait()
            lanes = lax.iota(jnp.int32, 16)
            zero = jnp.zeros((16,), jnp.int32)
            bv = zero + b
            for g in range(K // 16):
                pkv = pkl[pl.ds(i * K + g * 16, 16)]
                dlv = lax.shift_right_logical(pkv, 14)
                ev = lanes + (g * 16)

                def fbody(fo, c2):
                    fz = zero + fo * 16
                    for ff in range(16):
                        fv = fz + ff
                        v = plsc.load_gather(rows, [bv, ev, fv])
                        plsc.addupdate_scatter(acc, [dlv, fv], v)
                    return c2

                lax.fori_loop(0, H // 16, fbody, 0)

            return carry

        lax.fori_loop(0, nch, chunk, 0)
        pltpu.sync_copy(acc.at[pl.ds(0, PT)], out_hbm.at[pl.ds(lo, PT)])

    return layer(y, pkl, cnts)


def _k1_body(x_ref, w0_ref, deg_ref, y0_ref, dinv_ref):
    dinv = lax.rsqrt(deg_ref[...] + 1.0)
    xw = jnp.dot(x_ref[...], w0_ref[...], preferred_element_type=jnp.float32)
    y0_ref[...] = xw * dinv
    dinv_ref[...] = dinv


def _k1(x_pad, W0, deg):
    R = NP // 8
    return pl.pallas_call(
        _k1_body,
        out_shape=(
            jax.ShapeDtypeStruct((NP, H), jnp.float32),
            jax.ShapeDtypeStruct((NP, 1), jnp.float32),
        ),
        grid=(8,),
        in_specs=[
            pl.BlockSpec((R, D_IN), lambda i: (i, 0)),
            pl.BlockSpec((D_IN, H), lambda i: (0, 0)),
            pl.BlockSpec((R, 1), lambda i: (i, 0)),
        ],
        out_specs=(
            pl.BlockSpec((R, H), lambda i: (i, 0)),
            pl.BlockSpec((R, 1), lambda i: (i, 0)),
        ),
    )(x_pad, W0, deg.reshape(NP, 1))


def _elu(x):
    return jnp.where(x > 0, x, jnp.exp(jnp.minimum(x, 0.0)) - 1.0)


def _mid_body(s_ref, y_ref, res_ref, dinv_ref, w_ref, b_ref, g_ref, be_ref,
              h_ref, ynext_ref, *, residual):
    dinv = dinv_ref[...]
    conv = dinv * (s_ref[...] + y_ref[...]) + b_ref[...]
    if residual:
        conv = conv + res_ref[...]
    h = _elu(_BN_C * g_ref[...] * conv + be_ref[...])
    h_ref[...] = h
    ynext_ref[...] = jnp.dot(h, w_ref[...], preferred_element_type=jnp.float32) * dinv


def _k_mid(S, y, resid, dinv, Wn, b, g, be, residual):
    R = NP // 8
    return pl.pallas_call(
        functools.partial(_mid_body, residual=residual),
        out_shape=(
            jax.ShapeDtypeStruct((NP, H), jnp.float32),
            jax.ShapeDtypeStruct((NP, H), jnp.float32),
        ),
        grid=(8,),
        in_specs=[
            pl.BlockSpec((R, H), lambda i: (i, 0)),
            pl.BlockSpec((R, H), lambda i: (i, 0)),
            pl.BlockSpec((R, H), lambda i: (i, 0)),
            pl.BlockSpec((R, 1), lambda i: (i, 0)),
            pl.BlockSpec((H, H), lambda i: (0, 0)),
            pl.BlockSpec((H,), lambda i: (0,)),
            pl.BlockSpec((H,), lambda i: (0,)),
            pl.BlockSpec((H,), lambda i: (0,)),
        ],
        out_specs=(
            pl.BlockSpec((R, H), lambda i: (i, 0)),
            pl.BlockSpec((R, H), lambda i: (i, 0)),
        ),
    )(S, y, resid, dinv, Wn, b, g, be)


def _k4_body(s_ref, y_ref, res_ref, dinv_ref, b_ref, wc1_ref, bc1_ref,
             wc2_ref, bc2_ref, logp_ref, h2_ref):
    dinv = dinv_ref[...]
    h2 = dinv * (s_ref[...] + y_ref[...]) + b_ref[...] + res_ref[...]
    h2_ref[...] = h2
    z = _elu(jnp.dot(h2, wc1_ref[...], preferred_element_type=jnp.float32)
             + bc1_ref[...])
    logits = jnp.dot(z, wc2_ref[...], preferred_element_type=jnp.float32)
    logits = logits + bc2_ref[...]
    m = jnp.max(logits, axis=-1, keepdims=True)
    lse = jnp.log(jnp.sum(jnp.exp(logits - m), axis=-1, keepdims=True)) + m
    logp_ref[...] = logits - lse


def _k4(S2, y2, h1, dinv, b2, Wc1, bc1, Wc2p, bc2p):
    R = NP // 8
    return pl.pallas_call(
        _k4_body,
        out_shape=(
            jax.ShapeDtypeStruct((NP, 128), jnp.float32),
            jax.ShapeDtypeStruct((NP, H), jnp.float32),
        ),
        grid=(8,),
        in_specs=[
            pl.BlockSpec((R, H), lambda i: (i, 0)),
            pl.BlockSpec((R, H), lambda i: (i, 0)),
            pl.BlockSpec((R, H), lambda i: (i, 0)),
            pl.BlockSpec((R, 1), lambda i: (i, 0)),
            pl.BlockSpec((H,), lambda i: (0,)),
            pl.BlockSpec((H, H // 2), lambda i: (0, 0)),
            pl.BlockSpec((H // 2,), lambda i: (0,)),
            pl.BlockSpec((H // 2, 128), lambda i: (0, 0)),
            pl.BlockSpec((128,), lambda i: (0,)),
        ],
        out_specs=(
            pl.BlockSpec((R, 128), lambda i: (i, 0)),
            pl.BlockSpec((R, H), lambda i: (i, 0)),
        ),
    )(S2, y2, h1, dinv, b2, Wc1, bc1, Wc2p, bc2p)


def kernel(x, edgeIndex, W0, b0, W1, b1, W2, b2, g0, be0, g1, be1, Wc1, bc1, Wc2, bc2):
    x_pad = jnp.pad(x, ((0, NP - N), (0, 0)))
    Wc2p = jnp.zeros((H // 2, 128), jnp.float32).at[:, :C].set(Wc2)
    bc2p = jnp.full((128,), -1e9, jnp.float32).at[:C].set(bc2)

    deg, pkl, cnts = _sc_prologue(edgeIndex)
    y0, dinv = _k1(x_pad, W0, deg)
    S0 = _sc_layer(y0, pkl, cnts)
    h0, y1 = _k_mid(S0, y0, y0, dinv, W1, b0, g0, be0, residual=False)
    S1 = _sc_layer(y1, pkl, cnts)
    h1, y2 = _k_mid(S1, y1, h0, dinv, W2, b1, g1, be1, residual=True)
    S2 = _sc_layer(y2, pkl, cnts)
    logp, h2 = _k4(S2, y2, h1, dinv, b2, Wc1, bc1, Wc2p, bc2p)
    return logp[:N, :C], h2[:N]


# speedup vs baseline: 3.0644x; 3.0644x over previous
import functools

import jax
import jax.numpy as jnp
from jax import lax
from jax.experimental import pallas as pl
from jax.experimental.pallas import tpu as pltpu
from jax.experimental.pallas import tpu_sc as plsc

N = 10000
E = 320000
D_IN = 128
H = 256
C = 8

NP = 10240
NC = 2
NS = 16
TILES = NC * NS
PT = NP // TILES
CAP = 16384
K = 32
CH = 4000
_BN_C = 1.0 / (1.0 + 1e-5) ** 0.5


def _mesh():
    return plsc.VectorSubcoreMesh(core_axis_name="c", subcore_axis_name="s")


def _sc_prologue(edge):
    @functools.partial(
        pl.kernel,
        out_type=(
            jax.ShapeDtypeStruct((NP,), jnp.float32),
            jax.ShapeDtypeStruct((TILES, CAP), jnp.int32),
            jax.ShapeDtypeStruct((TILES, 16), jnp.int32),
        ),
        mesh=_mesh(),
        compiler_params=pltpu.CompilerParams(needs_layout_passes=False),
        scratch_types=[
            pltpu.VMEM((CH,), jnp.int32),
            pltpu.VMEM((CH,), jnp.int32),
            pltpu.VMEM((CAP,), jnp.int32),
            pltpu.VMEM((PT,), jnp.float32),
            pltpu.VMEM((16,), jnp.int32),
        ],
    )
    def prologue(edge_hbm, deg_hbm, pkl_hbm, cnt_hbm,
                 srcb, dstb, pkl, degl, cntv):
        cid = lax.axis_index("c")
        sid = lax.axis_index("s")
        wid = sid * NC + cid
        lo = wid * PT

        zf = jnp.zeros((16,), jnp.float32)

        def zdeg(i, carry):
            degl[pl.ds(i * 16, 16)] = zf
            return carry

        lax.fori_loop(0, PT // 16, zdeg, 0)

        lanes = lax.iota(jnp.int32, 16)

        def zl(i, carry):
            pkl[pl.ds(i * 16, 16)] = PT * 16384 + ((i * 16 + lanes) & 8191)
            return carry

        lax.fori_loop(0, CAP // 16, zl, 0)

        ones = jnp.ones((16,), jnp.float32)

        def chunk_body(ci, cnt):
            base = ci * CH
            pltpu.sync_copy(edge_hbm.at[pl.ds(base, CH)], srcb)
            pltpu.sync_copy(edge_hbm.at[pl.ds(E + base, CH)], dstb)

            def vec_body(j, cnt):
                sv = srcb[pl.ds(j * 16, 16)]
                dv = dstb[pl.ds(j * 16, 16)]
                dl = dv - lo
                m = (dl >= 0) & (dl < PT)
                mi = m.astype(jnp.int32)
                pos = cnt + plsc.cumsum(mi) - 1
                plsc.store_scatter(pkl, [pos], dl * 16384 + sv, mask=m)
                plsc.addupdate_scatter(degl, [dl], ones, mask=m)
                return pos[15] + 1

            return lax.fori_loop(0, CH // 16, vec_body, cnt)

        cnt = lax.fori_loop(0, E // CH, chunk_body, jnp.int32(0))
        cntv[...] = jnp.where(lax.iota(jnp.int32, 16) == 0, cnt, 0)
        pltpu.sync_copy(degl, deg_hbm.at[pl.ds(lo, PT)])
        pltpu.sync_copy(pkl, pkl_hbm.at[wid])
        pltpu.sync_copy(cntv, cnt_hbm.at[wid])

    return prologue(edge.reshape(2 * E))


def _sc_layer(y, pkl, cnts):
    @functools.partial(
        pl.kernel,
        out_type=jax.ShapeDtypeStruct((NP, H), jnp.float32),
        mesh=_mesh(),
        compiler_params=pltpu.CompilerParams(needs_layout_passes=False),
        scratch_types=[
            pltpu.VMEM((PT + 1, H), jnp.float32),
            pltpu.VMEM((CAP,), jnp.int32),
            pltpu.VMEM((2, K), jnp.int32),
            pltpu.VMEM((2, K, H), jnp.float32),
            pltpu.VMEM((16,), jnp.int32),
            pltpu.SemaphoreType.DMA((2,)),
        ],
    )
    def layer(y_hbm, pkl_hbm, cnt_hbm, out_hbm,
              acc, pkl, sidx, rows, cntv, sems):
        cid = lax.axis_index("c")
        sid = lax.axis_index("s")
        wid = sid * NC + cid
        lo = wid * PT

        pltpu.sync_copy(cnt_hbm.at[wid], cntv)
        pltpu.sync_copy(pkl_hbm.at[wid], pkl)
        cnt = jnp.sum(cntv[...])
        nch = (cnt + (K - 1)) // K

        zf = jnp.zeros((16,), jnp.float32)

        def zacc(r, carry):
            for u in range(H // 16):
                acc[r, pl.ds(u * 16, 16)] = zf
            return carry

        lax.fori_loop(0, PT + 1, zacc, 0)

        def issue(i, b):
            for g in range(K // 16):
                pkv = pkl[pl.ds(i * K + g * 16, 16)]
                sidx[b, pl.ds(g * 16, 16)] = pkv & 16383
            pltpu.async_copy(y_hbm.at[sidx.at[b]], rows.at[b], sems.at[b])

        @pl.when(nch > 0)
        def _():
            issue(0, 0)

        def chunk(i, carry):
            b = lax.rem(i, 2)

            @pl.when(i + 1 < nch)
            def _():
                issue(i + 1, 1 - b)

            pltpu.make_async_copy(
                y_hbm.at[sidx.at[b]], rows.at[b], sems.at[b]
            ).wait()
            for g in range(K // 16):
                pkv = pkl[pl.ds(i * K + g * 16, 16)]
                dlv = lax.shift_right_logical(pkv, 14)
                for u in range(16):
                    dl = dlv[u]
                    e = g * 16 + u
                    for f in range(H // 16):
                        a = acc[dl, pl.ds(f * 16, 16)]
                        v = rows[b, e, pl.ds(f * 16, 16)]
                        acc[dl, pl.ds(f * 16, 16)] = a + v

            return carry

        lax.fori_loop(0, nch, chunk, 0)
        pltpu.sync_copy(acc.at[pl.ds(0, PT)], out_hbm.at[pl.ds(lo, PT)])

    return layer(y, pkl, cnts)


def _k1_body(x_ref, w0_ref, deg_ref, y0_ref, dinv_ref):
    dinv = lax.rsqrt(deg_ref[...] + 1.0)
    xw = jnp.dot(x_ref[...], w0_ref[...], preferred_element_type=jnp.float32)
    y0_ref[...] = xw * dinv
    dinv_ref[...] = dinv


def _k1(x_pad, W0, deg):
    R = NP // 8
    return pl.pallas_call(
        _k1_body,
        out_shape=(
            jax.ShapeDtypeStruct((NP, H), jnp.float32),
            jax.ShapeDtypeStruct((NP, 1), jnp.float32),
        ),
        grid=(8,),
        in_specs=[
            pl.BlockSpec((R, D_IN), lambda i: (i, 0)),
            pl.BlockSpec((D_IN, H), lambda i: (0, 0)),
            pl.BlockSpec((R, 1), lambda i: (i, 0)),
        ],
        out_specs=(
            pl.BlockSpec((R, H), lambda i: (i, 0)),
            pl.BlockSpec((R, 1), lambda i: (i, 0)),
        ),
    )(x_pad, W0, deg.reshape(NP, 1))


def _elu(x):
    return jnp.where(x > 0, x, jnp.exp(jnp.minimum(x, 0.0)) - 1.0)


def _mid_body(s_ref, y_ref, res_ref, dinv_ref, w_ref, b_ref, g_ref, be_ref,
              h_ref, ynext_ref, *, residual):
    dinv = dinv_ref[...]
    conv = dinv * (s_ref[...] + y_ref[...]) + b_ref[...]
    if residual:
        conv = conv + res_ref[...]
    h = _elu(_BN_C * g_ref[...] * conv + be_ref[...])
    h_ref[...] = h
    ynext_ref[...] = jnp.dot(h, w_ref[...], preferred_element_type=jnp.float32) * dinv


def _k_mid(S, y, resid, dinv, Wn, b, g, be, residual):
    R = NP // 8
    return pl.pallas_call(
        functools.partial(_mid_body, residual=residual),
        out_shape=(
            jax.ShapeDtypeStruct((NP, H), jnp.float32),
            jax.ShapeDtypeStruct((NP, H), jnp.float32),
        ),
        grid=(8,),
        in_specs=[
            pl.BlockSpec((R, H), lambda i: (i, 0)),
            pl.BlockSpec((R, H), lambda i: (i, 0)),
            pl.BlockSpec((R, H), lambda i: (i, 0)),
            pl.BlockSpec((R, 1), lambda i: (i, 0)),
            pl.BlockSpec((H, H), lambda i: (0, 0)),
            pl.BlockSpec((H,), lambda i: (0,)),
            pl.BlockSpec((H,), lambda i: (0,)),
            pl.BlockSpec((H,), lambda i: (0,)),
        ],
        out_specs=(
            pl.BlockSpec((R, H), lambda i: (i, 0)),
            pl.BlockSpec((R, H), lambda i: (i, 0)),
        ),
    )(S, y, resid, dinv, Wn, b, g, be)


def _k4_body(s_ref, y_ref, res_ref, dinv_ref, b_ref, wc1_ref, bc1_ref,
             wc2_ref, bc2_ref, logp_ref, h2_ref):
    dinv = dinv_ref[...]
    h2 = dinv * (s_ref[...] + y_ref[...]) + b_ref[...] + res_ref[...]
    h2_ref[...] = h2
    z = _elu(jnp.dot(h2, wc1_ref[...], preferred_element_type=jnp.float32)
             + bc1_ref[...])
    logits = jnp.dot(z, wc2_ref[...], preferred_element_type=jnp.float32)
    logits = logits + bc2_ref[...]
    m = jnp.max(logits, axis=-1, keepdims=True)
    lse = jnp.log(jnp.sum(jnp.exp(logits - m), axis=-1, keepdims=True)) + m
    logp_ref[...] = logits - lse


def _k4(S2, y2, h1, dinv, b2, Wc1, bc1, Wc2p, bc2p):
    R = NP // 8
    return pl.pallas_call(
        _k4_body,
        out_shape=(
            jax.ShapeDtypeStruct((NP, 128), jnp.float32),
            jax.ShapeDtypeStruct((NP, H), jnp.float32),
        ),
        grid=(8,),
        in_specs=[
            pl.BlockSpec((R, H), lambda i: (i, 0)),
            pl.BlockSpec((R, H), lambda i: (i, 0)),
            pl.BlockSpec((R, H), lambda i: (i, 0)),
            pl.BlockSpec((R, 1), lambda i: (i, 0)),
            pl.BlockSpec((H,), lambda i: (0,)),
            pl.BlockSpec((H, H // 2), lambda i: (0, 0)),
            pl.BlockSpec((H // 2,), lambda i: (0,)),
            pl.BlockSpec((H // 2, 128), lambda i: (0, 0)),
            pl.BlockSpec((128,), lambda i: (0,)),
        ],
        out_specs=(
            pl.BlockSpec((R, 128), lambda i: (i, 0)),
            pl.BlockSpec((R, H), lambda i: (i, 0)),
        ),
    )(S2, y2, h1, dinv, b2, Wc1, bc1, Wc2p, bc2p)


def kernel(x, edgeIndex, W0, b0, W1, b1, W2, b2, g0, be0, g1, be1, Wc1, bc1, Wc2, bc2):
    x_pad = jnp.pad(x, ((0, NP - N), (0, 0)))
    Wc2p = jnp.zeros((H // 2, 128), jnp.float32).at[:, :C].set(Wc2)
    bc2p = jnp.full((128,), -1e9, jnp.float32).at[:C].set(bc2)

    deg, pkl, cnts = _sc_prologue(edgeIndex)
    y0, dinv = _k1(x_pad, W0, deg)
    S0 = _sc_layer(y0, pkl, cnts)
    h0, y1 = _k_mid(S0, y0, y0, dinv, W1, b0, g0, be0, residual=False)
    S1 = _sc_layer(y1, pkl, cnts)
    h1, y2 = _k_mid(S1, y1, h0, dinv, W2, b1, g1, be1, residual=True)
    S2 = _sc_layer(y2, pkl, cnts)
    logp, h2 = _k4(S2, y2, h1, dinv, b2, Wc1, bc1, Wc2p, bc2p)
    return logp[:N, :C], h2[:N]


# speedup vs baseline: 8.1137x; 2.6477x over previous
import functools

import jax
import jax.numpy as jnp
from jax import lax
from jax.experimental import pallas as pl
from jax.experimental.pallas import tpu as pltpu
from jax.experimental.pallas import tpu_sc as plsc

N = 10000
E = 320000
D_IN = 128
H = 256
C = 8

NP = 10240
NC = 2
NS = 16
TILES = NC * NS
PT = NP // TILES
CAP = 16384
K = 32
CH = 4000
_BN_C = 1.0 / (1.0 + 1e-5) ** 0.5


def _mesh():
    return plsc.VectorSubcoreMesh(core_axis_name="c", subcore_axis_name="s")


def _sc_prologue(edge):
    @functools.partial(
        pl.kernel,
        out_type=(
            jax.ShapeDtypeStruct((NP,), jnp.float32),
            jax.ShapeDtypeStruct((TILES, CAP), jnp.int32),
            jax.ShapeDtypeStruct((TILES, 16), jnp.int32),
        ),
        mesh=_mesh(),
        compiler_params=pltpu.CompilerParams(needs_layout_passes=False),
        scratch_types=[
            pltpu.VMEM((CH,), jnp.int32),
            pltpu.VMEM((CH,), jnp.int32),
            pltpu.VMEM((CAP,), jnp.int32),
            pltpu.VMEM((CAP,), jnp.int32),
            pltpu.VMEM((PT,), jnp.float32),
            pltpu.VMEM((PT,), jnp.int32),
            pltpu.VMEM((16,), jnp.int32),
        ],
    )
    def prologue(edge_hbm, deg_hbm, pkl_hbm, cnt_hbm,
                 srcb, dstb, pkl, spkl, degl, offt, cntv):
        cid = lax.axis_index("c")
        sid = lax.axis_index("s")
        wid = sid * NC + cid
        lo = wid * PT

        zf = jnp.zeros((16,), jnp.float32)

        def zdeg(i, carry):
            degl[pl.ds(i * 16, 16)] = zf
            return carry

        lax.fori_loop(0, PT // 16, zdeg, 0)

        lanes = lax.iota(jnp.int32, 16)

        def zl(i, carry):
            fill = PT * 16384 + ((i * 16 + lanes) & 8191)
            pkl[pl.ds(i * 16, 16)] = fill
            spkl[pl.ds(i * 16, 16)] = fill
            return carry

        lax.fori_loop(0, CAP // 16, zl, 0)

        ones = jnp.ones((16,), jnp.float32)

        def chunk_body(ci, cnt):
            base = ci * CH
            pltpu.sync_copy(edge_hbm.at[pl.ds(base, CH)], srcb)
            pltpu.sync_copy(edge_hbm.at[pl.ds(E + base, CH)], dstb)

            def vec_body(j, cnt):
                sv = srcb[pl.ds(j * 16, 16)]
                dv = dstb[pl.ds(j * 16, 16)]
                dl = dv - lo
                m = (dl >= 0) & (dl < PT)
                mi = m.astype(jnp.int32)
                pos = cnt + plsc.cumsum(mi) - 1
                plsc.store_scatter(pkl, [pos], dl * 16384 + sv, mask=m)
                plsc.addupdate_scatter(degl, [dl], ones, mask=m)
                return pos[15] + 1

            return lax.fori_loop(0, CH // 16, vec_body, cnt)

        cnt = lax.fori_loop(0, E // CH, chunk_body, jnp.int32(0))
        cntv[...] = jnp.where(lax.iota(jnp.int32, 16) == 0, cnt, 0)
        pltpu.sync_copy(degl, deg_hbm.at[pl.ds(lo, PT)])

        def poff(t, run):
            dv = degl[pl.ds(t * 16, 16)].astype(jnp.int32)
            cs = plsc.cumsum(dv)
            offt[pl.ds(t * 16, 16)] = (run + cs) - dv
            return run + cs[15]

        lax.fori_loop(0, PT // 16, poff, jnp.int32(0))

        onesi = jnp.ones((16,), jnp.int32)

        def p2(j, carry):
            pkv = pkl[pl.ds(j * 16, 16)]
            dlv = lax.shift_right_logical(pkv, 14)
            m = dlv < PT
            og = plsc.load_gather(offt, [dlv], mask=m)
            rk, _ = plsc.scan_count(dlv, mask=m)
            pos = (og + rk) - 1
            plsc.store_scatter(spkl, [pos], pkv, mask=m)
            plsc.addupdate_scatter(offt, [dlv], onesi, mask=m)
            return carry

        lax.fori_loop(0, (cnt + 15) // 16, p2, 0)
        pltpu.sync_copy(spkl, pkl_hbm.at[wid])
        pltpu.sync_copy(cntv, cnt_hbm.at[wid])

    return prologue(edge.reshape(2 * E))


def _sc_layer(y, pkl, cnts):
    @functools.partial(
        pl.kernel,
        out_type=jax.ShapeDtypeStruct((NP, H), jnp.float32),
        mesh=_mesh(),
        compiler_params=pltpu.CompilerParams(needs_layout_passes=False),
        scratch_types=[
            pltpu.VMEM((PT + 1, H), jnp.float32),
            pltpu.VMEM((CAP,), jnp.int32),
            pltpu.VMEM((2, K), jnp.int32),
            pltpu.VMEM((2, K, H), jnp.float32),
            pltpu.VMEM((16,), jnp.int32),
            pltpu.SemaphoreType.DMA((2,)),
        ],
    )
    def layer(y_hbm, pkl_hbm, cnt_hbm, out_hbm,
              acc, pkl, sidx, rows, cntv, sems):
        cid = lax.axis_index("c")
        sid = lax.axis_index("s")
        wid = sid * NC + cid
        lo = wid * PT

        pltpu.sync_copy(cnt_hbm.at[wid], cntv)
        pltpu.sync_copy(pkl_hbm.at[wid], pkl)
        cnt = jnp.sum(cntv[...])
        nch = (cnt + (K - 1)) // K

        zf = jnp.zeros((16,), jnp.float32)

        def zacc(r, carry):
            for u in range(H // 16):
                acc[r, pl.ds(u * 16, 16)] = zf
            return carry

        lax.fori_loop(0, PT + 1, zacc, 0)

        def issue(i, b):
            for g in range(K // 16):
                pkv = pkl[pl.ds(i * K + g * 16, 16)]
                sidx[b, pl.ds(g * 16, 16)] = pkv & 16383
            pltpu.async_copy(y_hbm.at[sidx.at[b]], rows.at[b], sems.at[b])

        @pl.when(nch > 0)
        def _():
            issue(0, 0)

        def chunk(i, carry):
            b = lax.rem(i, 2)

            @pl.when(i + 1 < nch)
            def _():
                issue(i + 1, 1 - b)

            pltpu.make_async_copy(
                y_hbm.at[sidx.at[b]], rows.at[b], sems.at[b]
            ).wait()
            FH = H // 16
            regs = [rows[b, 0, pl.ds(f * 16, 16)] for f in range(FH)]
            pkv0 = pkl[pl.ds(i * K, 16)]
            dlv0 = lax.shift_right_logical(pkv0, 14)
            prev = dlv0[0]
            for g in range(K // 16):
                dlv = (dlv0 if g == 0
                       else lax.shift_right_logical(
                           pkl[pl.ds(i * K + g * 16, 16)], 14))
                for u in range(16):
                    if g == 0 and u == 0:
                        continue
                    e = g * 16 + u
                    dl = dlv[u]
                    cond = dl != prev

                    @pl.when(cond)
                    def _(regs=regs, prev=prev):
                        for f in range(FH):
                            plsc.addupdate(
                                acc.at[prev, pl.ds(f * 16, 16)], regs[f])

                    for f in range(FH):
                        v = rows[b, e, pl.ds(f * 16, 16)]
                        regs[f] = jnp.where(cond, v, regs[f] + v)
                    prev = jnp.where(cond, dl, prev)
            for f in range(FH):
                plsc.addupdate(acc.at[prev, pl.ds(f * 16, 16)], regs[f])

            return carry

        lax.fori_loop(0, nch, chunk, 0)
        pltpu.sync_copy(acc.at[pl.ds(0, PT)], out_hbm.at[pl.ds(lo, PT)])

    return layer(y, pkl, cnts)


def _k1_body(x_ref, w0_ref, deg_ref, y0_ref, dinv_ref):
    dinv = lax.rsqrt(deg_ref[...] + 1.0)
    xw = jnp.dot(x_ref[...], w0_ref[...], preferred_element_type=jnp.float32)
    y0_ref[...] = xw * dinv
    dinv_ref[...] = dinv


def _k1(x_pad, W0, deg):
    R = NP // 8
    return pl.pallas_call(
        _k1_body,
        out_shape=(
            jax.ShapeDtypeStruct((NP, H), jnp.float32),
            jax.ShapeDtypeStruct((NP, 1), jnp.float32),
        ),
        grid=(8,),
        in_specs=[
            pl.BlockSpec((R, D_IN), lambda i: (i, 0)),
            pl.BlockSpec((D_IN, H), lambda i: (0, 0)),
            pl.BlockSpec((R, 1), lambda i: (i, 0)),
        ],
        out_specs=(
            pl.BlockSpec((R, H), lambda i: (i, 0)),
            pl.BlockSpec((R, 1), lambda i: (i, 0)),
        ),
    )(x_pad, W0, deg.reshape(NP, 1))


def _elu(x):
    return jnp.where(x > 0, x, jnp.exp(jnp.minimum(x, 0.0)) - 1.0)


def _mid_body(s_ref, y_ref, res_ref, dinv_ref, w_ref, b_ref, g_ref, be_ref,
              h_ref, ynext_ref, *, residual):
    dinv = dinv_ref[...]
    conv = dinv * (s_ref[...] + y_ref[...]) + b_ref[...]
    if residual:
        conv = conv + res_ref[...]
    h = _elu(_BN_C * g_ref[...] * conv + be_ref[...])
    h_ref[...] = h
    ynext_ref[...] = jnp.dot(h, w_ref[...], preferred_element_type=jnp.float32) * dinv


def _k_mid(S, y, resid, dinv, Wn, b, g, be, residual):
    R = NP // 8
    return pl.pallas_call(
        functools.partial(_mid_body, residual=residual),
        out_shape=(
            jax.ShapeDtypeStruct((NP, H), jnp.float32),
            jax.ShapeDtypeStruct((NP, H), jnp.float32),
        ),
        grid=(8,),
        in_specs=[
            pl.BlockSpec((R, H), lambda i: (i, 0)),
            pl.BlockSpec((R, H), lambda i: (i, 0)),
            pl.BlockSpec((R, H), lambda i: (i, 0)),
            pl.BlockSpec((R, 1), lambda i: (i, 0)),
            pl.BlockSpec((H, H), lambda i: (0, 0)),
            pl.BlockSpec((H,), lambda i: (0,)),
            pl.BlockSpec((H,), lambda i: (0,)),
            pl.BlockSpec((H,), lambda i: (0,)),
        ],
        out_specs=(
            pl.BlockSpec((R, H), lambda i: (i, 0)),
            pl.BlockSpec((R, H), lambda i: (i, 0)),
        ),
    )(S, y, resid, dinv, Wn, b, g, be)


def _k4_body(s_ref, y_ref, res_ref, dinv_ref, b_ref, wc1_ref, bc1_ref,
             wc2_ref, bc2_ref, logp_ref, h2_ref):
    dinv = dinv_ref[...]
    h2 = dinv * (s_ref[...] + y_ref[...]) + b_ref[...] + res_ref[...]
    h2_ref[...] = h2
    z = _elu(jnp.dot(h2, wc1_ref[...], preferred_element_type=jnp.float32)
             + bc1_ref[...])
    logits = jnp.dot(z, wc2_ref[...], preferred_element_type=jnp.float32)
    logits = logits + bc2_ref[...]
    m = jnp.max(logits, axis=-1, keepdims=True)
    lse = jnp.log(jnp.sum(jnp.exp(logits - m), axis=-1, keepdims=True)) + m
    logp_ref[...] = logits - lse


def _k4(S2, y2, h1, dinv, b2, Wc1, bc1, Wc2p, bc2p):
    R = NP // 8
    return pl.pallas_call(
        _k4_body,
        out_shape=(
            jax.ShapeDtypeStruct((NP, 128), jnp.float32),
            jax.ShapeDtypeStruct((NP, H), jnp.float32),
        ),
        grid=(8,),
        in_specs=[
            pl.BlockSpec((R, H), lambda i: (i, 0)),
            pl.BlockSpec((R, H), lambda i: (i, 0)),
            pl.BlockSpec((R, H), lambda i: (i, 0)),
            pl.BlockSpec((R, 1), lambda i: (i, 0)),
            pl.BlockSpec((H,), lambda i: (0,)),
            pl.BlockSpec((H, H // 2), lambda i: (0, 0)),
            pl.BlockSpec((H // 2,), lambda i: (0,)),
            pl.BlockSpec((H // 2, 128), lambda i: (0, 0)),
            pl.BlockSpec((128,), lambda i: (0,)),
        ],
        out_specs=(
            pl.BlockSpec((R, 128), lambda i: (i, 0)),
            pl.BlockSpec((R, H), lambda i: (i, 0)),
        ),
    )(S2, y2, h1, dinv, b2, Wc1, bc1, Wc2p, bc2p)


def kernel(x, edgeIndex, W0, b0, W1, b1, W2, b2, g0, be0, g1, be1, Wc1, bc1, Wc2, bc2):
    x_pad = jnp.pad(x, ((0, NP - N), (0, 0)))
    Wc2p = jnp.zeros((H // 2, 128), jnp.float32).at[:, :C].set(Wc2)
    bc2p = jnp.full((128,), -1e9, jnp.float32).at[:C].set(bc2)

    deg, pkl, cnts = _sc_prologue(edgeIndex)
    y0, dinv = _k1(x_pad, W0, deg)
    S0 = _sc_layer(y0, pkl, cnts)
    h0, y1 = _k_mid(S0, y0, y0, dinv, W1, b0, g0, be0, residual=False)
    S1 = _sc_layer(y1, pkl, cnts)
    h1, y2 = _k_mid(S1, y1, h0, dinv, W2, b1, g1, be1, residual=True)
    S2 = _sc_layer(y2, pkl, cnts)
    logp, h2 = _k4(S2, y2, h1, dinv, b2, Wc1, bc1, Wc2p, bc2p)
    return logp[:N, :C], h2[:N]


# speedup vs baseline: 8.7529x; 1.0788x over previous
import functools

import jax
import jax.numpy as jnp
from jax import lax
from jax.experimental import pallas as pl
from jax.experimental.pallas import tpu as pltpu
from jax.experimental.pallas import tpu_sc as plsc

N = 10000
E = 320000
D_IN = 128
H = 256
C = 8

NP = 10240
NC = 2
NS = 16
TILES = NC * NS
PT = NP // TILES
CAP = 16384
K = 32
CH = 4000
_BN_C = 1.0 / (1.0 + 1e-5) ** 0.5


def _mesh():
    return plsc.VectorSubcoreMesh(core_axis_name="c", subcore_axis_name="s")


def _sc_prologue(edge):
    @functools.partial(
        pl.kernel,
        out_type=(
            jax.ShapeDtypeStruct((NP,), jnp.float32),
            jax.ShapeDtypeStruct((TILES, CAP), jnp.int32),
            jax.ShapeDtypeStruct((TILES, 16), jnp.int32),
        ),
        mesh=_mesh(),
        compiler_params=pltpu.CompilerParams(needs_layout_passes=False),
        scratch_types=[
            pltpu.VMEM((CH,), jnp.int32),
            pltpu.VMEM((CH,), jnp.int32),
            pltpu.VMEM((CAP,), jnp.int32),
            pltpu.VMEM((CAP,), jnp.int32),
            pltpu.VMEM((PT,), jnp.float32),
            pltpu.VMEM((PT,), jnp.int32),
            pltpu.VMEM((16,), jnp.int32),
        ],
    )
    def prologue(edge_hbm, deg_hbm, pkl_hbm, cnt_hbm,
                 srcb, dstb, pkl, spkl, degl, offt, cntv):
        cid = lax.axis_index("c")
        sid = lax.axis_index("s")
        wid = sid * NC + cid
        lo = wid * PT

        zf = jnp.zeros((16,), jnp.float32)

        def zdeg(i, carry):
            degl[pl.ds(i * 16, 16)] = zf
            return carry

        lax.fori_loop(0, PT // 16, zdeg, 0)

        lanes = lax.iota(jnp.int32, 16)

        def zl(i, carry):
            fill = PT * 16384 + ((i * 16 + lanes) & 8191)
            pkl[pl.ds(i * 16, 16)] = fill
            spkl[pl.ds(i * 16, 16)] = fill
            return carry

        lax.fori_loop(0, CAP // 16, zl, 0)

        ones = jnp.ones((16,), jnp.float32)
        CAPH = CAP // 2
        CHH = CH // 2

        def chunk_body(ci, carry):
            base = ci * CH
            pltpu.sync_copy(edge_hbm.at[pl.ds(base, CH)], srcb)
            pltpu.sync_copy(edge_hbm.at[pl.ds(E + base, CH)], dstb)

            def vec_body(j, carry):
                ca, cb = carry
                sva = srcb[pl.ds(j * 16, 16)]
                dva = dstb[pl.ds(j * 16, 16)]
                svb = srcb[pl.ds(CHH + j * 16, 16)]
                dvb = dstb[pl.ds(CHH + j * 16, 16)]
                dla = dva - lo
                dlb = dvb - lo
                ma = (dla >= 0) & (dla < PT)
                mb = (dlb >= 0) & (dlb < PT)
                posa = ca + plsc.cumsum(ma.astype(jnp.int32)) - 1
                posb = cb + plsc.cumsum(mb.astype(jnp.int32)) - 1
                plsc.store_scatter(pkl, [posa], dla * 16384 + sva, mask=ma)
                plsc.store_scatter(pkl, [CAPH + posb], dlb * 16384 + svb,
                                   mask=mb)
                plsc.addupdate_scatter(degl, [dla], ones, mask=ma)
                plsc.addupdate_scatter(degl, [dlb], ones, mask=mb)
                return (posa[15] + 1, posb[15] + 1)

            return lax.fori_loop(0, CHH // 16, vec_body, carry)

        cnta, cntb = lax.fori_loop(0, E // CH, chunk_body,
                                   (jnp.int32(0), jnp.int32(0)))
        cnt = cnta + cntb
        cntv[...] = jnp.where(lax.iota(jnp.int32, 16) == 0, cnt, 0)
        pltpu.sync_copy(degl, deg_hbm.at[pl.ds(lo, PT)])

        def poff(t, run):
            dv = degl[pl.ds(t * 16, 16)].astype(jnp.int32)
            cs = plsc.cumsum(dv)
            offt[pl.ds(t * 16, 16)] = (run + cs) - dv
            return run + cs[15]

        lax.fori_loop(0, PT // 16, poff, jnp.int32(0))

        onesi = jnp.ones((16,), jnp.int32)

        def p2_seg(segbase):
            def p2(j, carry):
                pkv = pkl[pl.ds(segbase + j * 16, 16)]
                dlv = lax.shift_right_logical(pkv, 14)
                m = dlv < PT
                og = plsc.load_gather(offt, [dlv], mask=m)
                rk, _ = plsc.scan_count(dlv, mask=m)
                pos = (og + rk) - 1
                plsc.store_scatter(spkl, [pos], pkv, mask=m)
                plsc.addupdate_scatter(offt, [dlv], onesi, mask=m)
                return carry
            return p2

        lax.fori_loop(0, (cnta + 15) // 16, p2_seg(0), 0)
        lax.fori_loop(0, (cntb + 15) // 16, p2_seg(CAP // 2), 0)
        pltpu.sync_copy(spkl, pkl_hbm.at[wid])
        pltpu.sync_copy(cntv, cnt_hbm.at[wid])

    return prologue(edge.reshape(2 * E))


def _sc_layer(y, pkl, cnts):
    @functools.partial(
        pl.kernel,
        out_type=jax.ShapeDtypeStruct((NP, H), jnp.float32),
        mesh=_mesh(),
        compiler_params=pltpu.CompilerParams(needs_layout_passes=False),
        scratch_types=[
            pltpu.VMEM((PT + 1, H), jnp.float32),
            pltpu.VMEM((CAP,), jnp.int32),
            pltpu.VMEM((2, K), jnp.int32),
            pltpu.VMEM((2, K, H), jnp.float32),
            pltpu.VMEM((16,), jnp.int32),
            pltpu.SemaphoreType.DMA((2,)),
        ],
    )
    def layer(y_hbm, pkl_hbm, cnt_hbm, out_hbm,
              acc, pkl, sidx, rows, cntv, sems):
        cid = lax.axis_index("c")
        sid = lax.axis_index("s")
        wid = sid * NC + cid
        lo = wid * PT

        pltpu.sync_copy(cnt_hbm.at[wid], cntv)
        pltpu.sync_copy(pkl_hbm.at[wid], pkl)
        cnt = jnp.sum(cntv[...])
        nch = (cnt + (K - 1)) // K

        zf = jnp.zeros((16,), jnp.float32)

        def zacc(r, carry):
            for u in range(H // 16):
                acc[r, pl.ds(u * 16, 16)] = zf
            return carry

        lax.fori_loop(0, PT + 1, zacc, 0)

        def issue(i, b):
            for g in range(K // 16):
                pkv = pkl[pl.ds(i * K + g * 16, 16)]
                sidx[b, pl.ds(g * 16, 16)] = pkv & 16383
            pltpu.async_copy(y_hbm.at[sidx.at[b]], rows.at[b], sems.at[b])

        @pl.when(nch > 0)
        def _():
            issue(0, 0)

        def chunk(i, carry):
            b = lax.rem(i, 2)

            @pl.when(i + 1 < nch)
            def _():
                issue(i + 1, 1 - b)

            pltpu.make_async_copy(
                y_hbm.at[sidx.at[b]], rows.at[b], sems.at[b]
            ).wait()
            FH = H // 16
            regs = [rows[b, 0, pl.ds(f * 16, 16)] for f in range(FH)]
            pkv0 = pkl[pl.ds(i * K, 16)]
            dlv0 = lax.shift_right_logical(pkv0, 14)
            prev = dlv0[0]
            for g in range(K // 16):
                dlv = (dlv0 if g == 0
                       else lax.shift_right_logical(
                           pkl[pl.ds(i * K + g * 16, 16)], 14))
                for u in range(16):
                    if g == 0 and u == 0:
                        continue
                    e = g * 16 + u
                    dl = dlv[u]
                    cond = dl != prev

                    @pl.when(cond)
                    def _(regs=regs, prev=prev):
                        for f in range(FH):
                            plsc.addupdate(
                                acc.at[prev, pl.ds(f * 16, 16)], regs[f])

                    for f in range(FH):
                        v = rows[b, e, pl.ds(f * 16, 16)]
                        regs[f] = jnp.where(cond, v, regs[f] + v)
                    prev = jnp.where(cond, dl, prev)
            for f in range(FH):
                plsc.addupdate(acc.at[prev, pl.ds(f * 16, 16)], regs[f])

            return carry

        lax.fori_loop(0, nch, chunk, 0)
        pltpu.sync_copy(acc.at[pl.ds(0, PT)], out_hbm.at[pl.ds(lo, PT)])

    return layer(y, pkl, cnts)


def _k1_body(x_ref, w0_ref, deg_ref, y0_ref, dinv_ref):
    dinv = lax.rsqrt(deg_ref[...] + 1.0)
    xw = jnp.dot(x_ref[...], w0_ref[...], preferred_element_type=jnp.float32)
    y0_ref[...] = xw * dinv
    dinv_ref[...] = dinv


def _k1(x_pad, W0, deg):
    R = NP // 8
    return pl.pallas_call(
        _k1_body,
        out_shape=(
            jax.ShapeDtypeStruct((NP, H), jnp.float32),
            jax.ShapeDtypeStruct((NP, 1), jnp.float32),
        ),
        grid=(8,),
        in_specs=[
            pl.BlockSpec((R, D_IN), lambda i: (i, 0)),
            pl.BlockSpec((D_IN, H), lambda i: (0, 0)),
            pl.BlockSpec((R, 1), lambda i: (i, 0)),
        ],
        out_specs=(
            pl.BlockSpec((R, H), lambda i: (i, 0)),
            pl.BlockSpec((R, 1), lambda i: (i, 0)),
        ),
    )(x_pad, W0, deg.reshape(NP, 1))


def _elu(x):
    return jnp.where(x > 0, x, jnp.exp(jnp.minimum(x, 0.0)) - 1.0)


def _mid_body(s_ref, y_ref, res_ref, dinv_ref, w_ref, b_ref, g_ref, be_ref,
              h_ref, ynext_ref, *, residual):
    dinv = dinv_ref[...]
    conv = dinv * (s_ref[...] + y_ref[...]) + b_ref[...]
    if residual:
        conv = conv + res_ref[...]
    h = _elu(_BN_C * g_ref[...] * conv + be_ref[...])
    h_ref[...] = h
    ynext_ref[...] = jnp.dot(h, w_ref[...], preferred_element_type=jnp.float32) * dinv


def _k_mid(S, y, resid, dinv, Wn, b, g, be, residual):
    R = NP // 8
    return pl.pallas_call(
        functools.partial(_mid_body, residual=residual),
        out_shape=(
            jax.ShapeDtypeStruct((NP, H), jnp.float32),
            jax.ShapeDtypeStruct((NP, H), jnp.float32),
        ),
        grid=(8,),
        in_specs=[
            pl.BlockSpec((R, H), lambda i: (i, 0)),
            pl.BlockSpec((R, H), lambda i: (i, 0)),
            pl.BlockSpec((R, H), lambda i: (i, 0)),
            pl.BlockSpec((R, 1), lambda i: (i, 0)),
            pl.BlockSpec((H, H), lambda i: (0, 0)),
            pl.BlockSpec((H,), lambda i: (0,)),
            pl.BlockSpec((H,), lambda i: (0,)),
            pl.BlockSpec((H,), lambda i: (0,)),
        ],
        out_specs=(
            pl.BlockSpec((R, H), lambda i: (i, 0)),
            pl.BlockSpec((R, H), lambda i: (i, 0)),
        ),
    )(S, y, resid, dinv, Wn, b, g, be)


def _k4_body(s_ref, y_ref, res_ref, dinv_ref, b_ref, wc1_ref, bc1_ref,
             wc2_ref, bc2_ref, logp_ref, h2_ref):
    dinv = dinv_ref[...]
    h2 = dinv * (s_ref[...] + y_ref[...]) + b_ref[...] + res_ref[...]
    h2_ref[...] = h2
    z = _elu(jnp.dot(h2, wc1_ref[...], preferred_element_type=jnp.float32)
             + bc1_ref[...])
    logits = jnp.dot(z, wc2_ref[...], preferred_element_type=jnp.float32)
    logits = logits + bc2_ref[...]
    m = jnp.max(logits, axis=-1, keepdims=True)
    lse = jnp.log(jnp.sum(jnp.exp(logits - m), axis=-1, keepdims=True)) + m
    logp_ref[...] = logits - lse


def _k4(S2, y2, h1, dinv, b2, Wc1, bc1, Wc2p, bc2p):
    R = NP // 8
    return pl.pallas_call(
        _k4_body,
        out_shape=(
            jax.ShapeDtypeStruct((NP, 128), jnp.float32),
            jax.ShapeDtypeStruct((NP, H), jnp.float32),
        ),
        grid=(8,),
        in_specs=[
            pl.BlockSpec((R, H), lambda i: (i, 0)),
            pl.BlockSpec((R, H), lambda i: (i, 0)),
            pl.BlockSpec((R, H), lambda i: (i, 0)),
            pl.BlockSpec((R, 1), lambda i: (i, 0)),
            pl.BlockSpec((H,), lambda i: (0,)),
            pl.BlockSpec((H, H // 2), lambda i: (0, 0)),
            pl.BlockSpec((H // 2,), lambda i: (0,)),
            pl.BlockSpec((H // 2, 128), lambda i: (0, 0)),
            pl.BlockSpec((128,), lambda i: (0,)),
        ],
        out_specs=(
            pl.BlockSpec((R, 128), lambda i: (i, 0)),
            pl.BlockSpec((R, H), lambda i: (i, 0)),
        ),
    )(S2, y2, h1, dinv, b2, Wc1, bc1, Wc2p, bc2p)


def kernel(x, edgeIndex, W0, b0, W1, b1, W2, b2, g0, be0, g1, be1, Wc1, bc1, Wc2, bc2):
    x_pad = jnp.pad(x, ((0, NP - N), (0, 0)))
    Wc2p = jnp.zeros((H // 2, 128), jnp.float32).at[:, :C].set(Wc2)
    bc2p = jnp.full((128,), -1e9, jnp.float32).at[:C].set(bc2)

    deg, pkl, cnts = _sc_prologue(edgeIndex)
    y0, dinv = _k1(x_pad, W0, deg)
    S0 = _sc_layer(y0, pkl, cnts)
    h0, y1 = _k_mid(S0, y0, y0, dinv, W1, b0, g0, be0, residual=False)
    S1 = _sc_layer(y1, pkl, cnts)
    h1, y2 = _k_mid(S1, y1, h0, dinv, W2, b1, g1, be1, residual=True)
    S2 = _sc_layer(y2, pkl, cnts)
    logp, h2 = _k4(S2, y2, h1, dinv, b2, Wc1, bc1, Wc2p, bc2p)
    return logp[:N, :C], h2[:N]


# speedup vs baseline: 10.0215x; 1.1449x over previous
import functools

import jax
import jax.numpy as jnp
from jax import lax
from jax.experimental import pallas as pl
from jax.experimental.pallas import tpu as pltpu
from jax.experimental.pallas import tpu_sc as plsc

N = 10000
E = 320000
D_IN = 128
H = 256
C = 8

NP = 10240
NC = 2
NS = 16
TILES = NC * NS
PT = NP // TILES
CAP = 16384
K = 32
CH = 4000
_BN_C = 1.0 / (1.0 + 1e-5) ** 0.5


def _mesh():
    return plsc.VectorSubcoreMesh(core_axis_name="c", subcore_axis_name="s")


def _sc_prologue(edge):
    @functools.partial(
        pl.kernel,
        out_type=(
            jax.ShapeDtypeStruct((NP,), jnp.float32),
            jax.ShapeDtypeStruct((TILES, CAP), jnp.int32),
            jax.ShapeDtypeStruct((TILES, 16), jnp.int32),
        ),
        mesh=_mesh(),
        compiler_params=pltpu.CompilerParams(needs_layout_passes=False),
        scratch_types=[
            pltpu.VMEM((CH,), jnp.int32),
            pltpu.VMEM((CH,), jnp.int32),
            pltpu.VMEM((CAP,), jnp.int32),
            pltpu.VMEM((CAP,), jnp.int32),
            pltpu.VMEM((PT,), jnp.float32),
            pltpu.VMEM((PT,), jnp.int32),
            pltpu.VMEM((16,), jnp.int32),
        ],
    )
    def prologue(edge_hbm, deg_hbm, pkl_hbm, cnt_hbm,
                 srcb, dstb, pkl, spkl, degl, offt, cntv):
        cid = lax.axis_index("c")
        sid = lax.axis_index("s")
        wid = sid * NC + cid
        lo = wid * PT

        zf = jnp.zeros((16,), jnp.float32)

        def zdeg(i, carry):
            degl[pl.ds(i * 16, 16)] = zf
            return carry

        lax.fori_loop(0, PT // 16, zdeg, 0)

        lanes = lax.iota(jnp.int32, 16)

        def zl(i, carry):
            fill = PT * 16384 + ((i * 16 + lanes) & 8191)
            pkl[pl.ds(i * 16, 16)] = fill
            spkl[pl.ds(i * 16, 16)] = fill
            return carry

        lax.fori_loop(0, CAP // 16, zl, 0)

        ones = jnp.ones((16,), jnp.float32)
        CAPH = CAP // 2
        CHH = CH // 2

        def chunk_body(ci, carry):
            base = ci * CH
            pltpu.sync_copy(edge_hbm.at[pl.ds(base, CH)], srcb)
            pltpu.sync_copy(edge_hbm.at[pl.ds(E + base, CH)], dstb)

            def vec_body(j, carry):
                ca, cb = carry
                sva = srcb[pl.ds(j * 16, 16)]
                dva = dstb[pl.ds(j * 16, 16)]
                svb = srcb[pl.ds(CHH + j * 16, 16)]
                dvb = dstb[pl.ds(CHH + j * 16, 16)]
                dla = dva - lo
                dlb = dvb - lo
                ma = (dla >= 0) & (dla < PT)
                mb = (dlb >= 0) & (dlb < PT)
                posa = ca + plsc.cumsum(ma.astype(jnp.int32)) - 1
                posb = cb + plsc.cumsum(mb.astype(jnp.int32)) - 1
                plsc.store_scatter(pkl, [posa], dla * 16384 + sva, mask=ma)
                plsc.store_scatter(pkl, [CAPH + posb], dlb * 16384 + svb,
                                   mask=mb)
                plsc.addupdate_scatter(degl, [dla], ones, mask=ma)
                plsc.addupdate_scatter(degl, [dlb], ones, mask=mb)
                return (posa[15] + 1, posb[15] + 1)

            return lax.fori_loop(0, CHH // 16, vec_body, carry)

        cnta, cntb = lax.fori_loop(0, E // CH, chunk_body,
                                   (jnp.int32(0), jnp.int32(0)))
        cnt = cnta + cntb
        cntv[...] = jnp.where(lax.iota(jnp.int32, 16) == 0, cnt, 0)
        pltpu.sync_copy(degl, deg_hbm.at[pl.ds(lo, PT)])

        def poff(t, run):
            dv = degl[pl.ds(t * 16, 16)].astype(jnp.int32)
            cs = plsc.cumsum(dv)
            offt[pl.ds(t * 16, 16)] = (run + cs) - dv
            return run + cs[15]

        lax.fori_loop(0, PT // 16, poff, jnp.int32(0))

        onesi = jnp.ones((16,), jnp.int32)

        def p2_seg(segbase):
            def p2(j, carry):
                pkv = pkl[pl.ds(segbase + j * 16, 16)]
                dlv = lax.shift_right_logical(pkv, 14)
                m = dlv < PT
                og = plsc.load_gather(offt, [dlv], mask=m)
                rk, _ = plsc.scan_count(dlv, mask=m)
                pos = (og + rk) - 1
                plsc.store_scatter(spkl, [pos], pkv, mask=m)
                plsc.addupdate_scatter(offt, [dlv], onesi, mask=m)
                return carry
            return p2

        lax.fori_loop(0, (cnta + 15) // 16, p2_seg(0), 0)
        lax.fori_loop(0, (cntb + 15) // 16, p2_seg(CAP // 2), 0)
        pltpu.sync_copy(spkl, pkl_hbm.at[wid])
        pltpu.sync_copy(cntv, cnt_hbm.at[wid])

    return prologue(edge.reshape(2 * E))


def _sc_layer(y, pkl, cnts):
    @functools.partial(
        pl.kernel,
        out_type=jax.ShapeDtypeStruct((NP, H), jnp.float32),
        mesh=_mesh(),
        compiler_params=pltpu.CompilerParams(needs_layout_passes=False),
        scratch_types=[
            pltpu.VMEM((PT + 1, H), jnp.float32),
            pltpu.VMEM((CAP,), jnp.int32),
            pltpu.VMEM((2, K), jnp.int32),
            pltpu.VMEM((2, K, H // 2), jnp.uint32),
            pltpu.VMEM((16,), jnp.int32),
            pltpu.SemaphoreType.DMA((2,)),
        ],
    )
    def layer(y_hbm, pkl_hbm, cnt_hbm, out_hbm,
              acc, pkl, sidx, rows, cntv, sems):
        cid = lax.axis_index("c")
        sid = lax.axis_index("s")
        wid = sid * NC + cid
        lo = wid * PT

        pltpu.sync_copy(cnt_hbm.at[wid], cntv)
        pltpu.sync_copy(pkl_hbm.at[wid], pkl)
        cnt = jnp.sum(cntv[...])
        nch = (cnt + (K - 1)) // K

        zf = jnp.zeros((16,), jnp.float32)

        def zacc(r, carry):
            for u in range(H // 16):
                acc[r, pl.ds(u * 16, 16)] = zf
            return carry

        lax.fori_loop(0, PT + 1, zacc, 0)

        def issue(i, b):
            for g in range(K // 16):
                pkv = pkl[pl.ds(i * K + g * 16, 16)]
                sidx[b, pl.ds(g * 16, 16)] = pkv & 16383
            pltpu.async_copy(y_hbm.at[sidx.at[b]], rows.at[b], sems.at[b])

        @pl.when(nch > 0)
        def _():
            issue(0, 0)

        def chunk(i, carry):
            b = lax.rem(i, 2)

            @pl.when(i + 1 < nch)
            def _():
                issue(i + 1, 1 - b)

            pltpu.make_async_copy(
                y_hbm.at[sidx.at[b]], rows.at[b], sems.at[b]
            ).wait()
            FH = H // 16

            def loadrow(e):
                out = [None] * FH
                for f in range(H // 32):
                    v = rows[b, e, pl.ds(f * 16, 16)]
                    vbf = plsc.bitcast(v, jnp.bfloat16)
                    va, vb = plsc.unpack(
                        vbf, format=plsc.PackFormat.INTERLEAVED,
                        preferred_element_type=jnp.float32)
                    out[f] = va
                    out[8 + f] = vb
                return out

            regs = loadrow(0)
            pkv0 = pkl[pl.ds(i * K, 16)]
            dlv0 = lax.shift_right_logical(pkv0, 14)
            prev = dlv0[0]
            for g in range(K // 16):
                dlv = (dlv0 if g == 0
                       else lax.shift_right_logical(
                           pkl[pl.ds(i * K + g * 16, 16)], 14))
                for u in range(16):
                    if g == 0 and u == 0:
                        continue
                    e = g * 16 + u
                    dl = dlv[u]
                    cond = dl != prev

                    @pl.when(cond)
                    def _(regs=regs, prev=prev):
                        for f in range(FH):
                            plsc.addupdate(
                                acc.at[prev, pl.ds(f * 16, 16)], regs[f])

                    vr = loadrow(e)
                    for f in range(FH):
                        regs[f] = jnp.where(cond, vr[f], regs[f] + vr[f])
                    prev = jnp.where(cond, dl, prev)
            for f in range(FH):
                plsc.addupdate(acc.at[prev, pl.ds(f * 16, 16)], regs[f])

            return carry

        lax.fori_loop(0, nch, chunk, 0)
        pltpu.sync_copy(acc.at[pl.ds(0, PT)], out_hbm.at[pl.ds(lo, PT)])

    return layer(y, pkl, cnts)


def _packcols(y):
    ya = y[:, : H // 2].astype(jnp.bfloat16)
    yb = y[:, H // 2:].astype(jnp.bfloat16)
    pa = lax.bitcast_convert_type(ya, jnp.uint16).astype(jnp.uint32)
    pb = lax.bitcast_convert_type(yb, jnp.uint16).astype(jnp.uint32)
    return pa | (pb << 16)


def _k1_body(x_ref, w0_ref, deg_ref, y0_ref, dinv_ref, y0p_ref):
    dinv = lax.rsqrt(deg_ref[...] + 1.0)
    xw = jnp.dot(x_ref[...], w0_ref[...], preferred_element_type=jnp.float32)
    y0 = xw * dinv
    y0_ref[...] = y0
    dinv_ref[...] = dinv
    y0p_ref[...] = _packcols(y0)


def _k1(x_pad, W0, deg):
    R = NP // 8
    return pl.pallas_call(
        _k1_body,
        out_shape=(
            jax.ShapeDtypeStruct((NP, H), jnp.float32),
            jax.ShapeDtypeStruct((NP, 1), jnp.float32),
            jax.ShapeDtypeStruct((NP, H // 2), jnp.uint32),
        ),
        grid=(8,),
        in_specs=[
            pl.BlockSpec((R, D_IN), lambda i: (i, 0)),
            pl.BlockSpec((D_IN, H), lambda i: (0, 0)),
            pl.BlockSpec((R, 1), lambda i: (i, 0)),
        ],
        out_specs=(
            pl.BlockSpec((R, H), lambda i: (i, 0)),
            pl.BlockSpec((R, 1), lambda i: (i, 0)),
            pl.BlockSpec((R, H // 2), lambda i: (i, 0)),
        ),
    )(x_pad, W0, deg.reshape(NP, 1))


def _elu(x):
    return jnp.where(x > 0, x, jnp.exp(jnp.minimum(x, 0.0)) - 1.0)


def _mid_body(s_ref, y_ref, res_ref, dinv_ref, w_ref, b_ref, g_ref, be_ref,
              h_ref, ynext_ref, ynextp_ref, *, residual):
    dinv = dinv_ref[...]
    conv = dinv * (s_ref[...] + y_ref[...]) + b_ref[...]
    if residual:
        conv = conv + res_ref[...]
    h = _elu(_BN_C * g_ref[...] * conv + be_ref[...])
    h_ref[...] = h
    ynext = jnp.dot(h, w_ref[...], preferred_element_type=jnp.float32) * dinv
    ynext_ref[...] = ynext
    ynextp_ref[...] = _packcols(ynext)


def _k_mid(S, y, resid, dinv, Wn, b, g, be, residual):
    R = NP // 8
    return pl.pallas_call(
        functools.partial(_mid_body, residual=residual),
        out_shape=(
            jax.ShapeDtypeStruct((NP, H), jnp.float32),
            jax.ShapeDtypeStruct((NP, H), jnp.float32),
            jax.ShapeDtypeStruct((NP, H // 2), jnp.uint32),
        ),
        grid=(8,),
        in_specs=[
            pl.BlockSpec((R, H), lambda i: (i, 0)),
            pl.BlockSpec((R, H), lambda i: (i, 0)),
            pl.BlockSpec((R, H), lambda i: (i, 0)),
            pl.BlockSpec((R, 1), lambda i: (i, 0)),
            pl.BlockSpec((H, H), lambda i: (0, 0)),
            pl.BlockSpec((H,), lambda i: (0,)),
            pl.BlockSpec((H,), lambda i: (0,)),
            pl.BlockSpec((H,), lambda i: (0,)),
        ],
        out_specs=(
            pl.BlockSpec((R, H), lambda i: (i, 0)),
            pl.BlockSpec((R, H), lambda i: (i, 0)),
            pl.BlockSpec((R, H // 2), lambda i: (i, 0)),
        ),
    )(S, y, resid, dinv, Wn, b, g, be)


def _k4_body(s_ref, y_ref, res_ref, dinv_ref, b_ref, wc1_ref, bc1_ref,
             wc2_ref, bc2_ref, logp_ref, h2_ref):
    dinv = dinv_ref[...]
    h2 = dinv * (s_ref[...] + y_ref[...]) + b_ref[...] + res_ref[...]
    h2_ref[...] = h2
    z = _elu(jnp.dot(h2, wc1_ref[...], preferred_element_type=jnp.float32)
             + bc1_ref[...])
    logits = jnp.dot(z, wc2_ref[...], preferred_element_type=jnp.float32)
    logits = logits + bc2_ref[...]
    m = jnp.max(logits, axis=-1, keepdims=True)
    lse = jnp.log(jnp.sum(jnp.exp(logits - m), axis=-1, keepdims=True)) + m
    logp_ref[...] = logits - lse


def _k4(S2, y2, h1, dinv, b2, Wc1, bc1, Wc2p, bc2p):
    R = NP // 8
    return pl.pallas_call(
        _k4_body,
        out_shape=(
            jax.ShapeDtypeStruct((NP, 128), jnp.float32),
            jax.ShapeDtypeStruct((NP, H), jnp.float32),
        ),
        grid=(8,),
        in_specs=[
            pl.BlockSpec((R, H), lambda i: (i, 0)),
            pl.BlockSpec((R, H), lambda i: (i, 0)),
            pl.BlockSpec((R, H), lambda i: (i, 0)),
            pl.BlockSpec((R, 1), lambda i: (i, 0)),
            pl.BlockSpec((H,), lambda i: (0,)),
            pl.BlockSpec((H, H // 2), lambda i: (0, 0)),
            pl.BlockSpec((H // 2,), lambda i: (0,)),
            pl.BlockSpec((H // 2, 128), lambda i: (0, 0)),
            pl.BlockSpec((128,), lambda i: (0,)),
        ],
        out_specs=(
            pl.BlockSpec((R, 128), lambda i: (i, 0)),
            pl.BlockSpec((R, H), lambda i: (i, 0)),
        ),
    )(S2, y2, h1, dinv, b2, Wc1, bc1, Wc2p, bc2p)


def kernel(x, edgeIndex, W0, b0, W1, b1, W2, b2, g0, be0, g1, be1, Wc1, bc1, Wc2, bc2):
    x_pad = jnp.pad(x, ((0, NP - N), (0, 0)))
    Wc2p = jnp.zeros((H // 2, 128), jnp.float32).at[:, :C].set(Wc2)
    bc2p = jnp.full((128,), -1e9, jnp.float32).at[:C].set(bc2)

    deg, pkl, cnts = _sc_prologue(edgeIndex)
    y0, dinv, y0p = _k1(x_pad, W0, deg)
    S0 = _sc_layer(y0p, pkl, cnts)
    h0, y1, y1p = _k_mid(S0, y0, y0, dinv, W1, b0, g0, be0, residual=False)
    S1 = _sc_layer(y1p, pkl, cnts)
    h1, y2, y2p = _k_mid(S1, y1, h0, dinv, W2, b1, g1, be1, residual=True)
    S2 = _sc_layer(y2p, pkl, cnts)
    logp, h2 = _k4(S2, y2, h1, dinv, b2, Wc1, bc1, Wc2p, bc2p)
    return logp[:N, :C], h2[:N]
